# Initial kernel scaffold; baseline (speedup 1.0000x reference)
import jax
import jax.numpy as jnp
from jax.experimental import pallas as pl


def kernel(x, edge_index, edge_attr, params):
    raise NotImplementedError("write your pallas kernel here")


# speedup vs baseline: 3.5698x; 3.5698x over previous
import functools

import jax
import jax.numpy as jnp
from jax.experimental import pallas as pl

N = 10000
E = 320000
F = 128
R = 5

ROW_BLK = 1000


def _rel_mm_body(x_ref, w_ref, o_ref):
    o_ref[0] = jnp.dot(x_ref[...], w_ref[0],
                       preferred_element_type=jnp.float32)


def rel_matmul(x, w_all):
    J, Din, Fo = w_all.shape
    n = x.shape[0]
    grid = (J, n // ROW_BLK)
    return pl.pallas_call(
        _rel_mm_body,
        grid=grid,
        in_specs=[
            pl.BlockSpec((ROW_BLK, Din), lambda j, i: (i, 0)),
            pl.BlockSpec((1, Din, Fo), lambda j, i: (j, 0, 0)),
        ],
        out_specs=pl.BlockSpec((1, ROW_BLK, Fo), lambda j, i: (j, i, 0)),
        out_shape=jax.ShapeDtypeStruct((J, n, Fo), jnp.float32),
    )(x, w_all)


def _epilogue_body(agg_ref, root_ref, s_ref, c_ref, a_ref, res_ref, o_ref):
    t = (agg_ref[...] + root_ref[...]) * s_ref[...] + c_ref[...]
    t = jnp.where(t >= 0, t, a_ref[0, 0] * t)
    o_ref[...] = t + res_ref[...]


def _epilogue_body_nores(agg_ref, root_ref, s_ref, c_ref, a_ref, o_ref):
    t = (agg_ref[...] + root_ref[...]) * s_ref[...] + c_ref[...]
    o_ref[...] = jnp.where(t >= 0, t, a_ref[0, 0] * t)


def conv_epilogue(agg, root, s, c, a, res=None):
    n = agg.shape[0]
    grid = (n // ROW_BLK,)
    row_spec = pl.BlockSpec((ROW_BLK, F), lambda i: (i, 0))
    vec_spec = pl.BlockSpec((1, F), lambda i: (0, 0))
    scal_spec = pl.BlockSpec((1, 1), lambda i: (0, 0))
    if res is None:
        return pl.pallas_call(
            _epilogue_body_nores,
            grid=grid,
            in_specs=[row_spec, row_spec, vec_spec, vec_spec, scal_spec],
            out_specs=row_spec,
            out_shape=jax.ShapeDtypeStruct((n, F), jnp.float32),
        )(agg, root, s, c, a)
    return pl.pallas_call(
        _epilogue_body,
        grid=grid,
        in_specs=[row_spec, row_spec, vec_spec, vec_spec, scal_spec, row_spec],
        out_specs=row_spec,
        out_shape=jax.ShapeDtypeStruct((n, F), jnp.float32),
    )(agg, root, s, c, a, res)


def _mm_bias_body(x_ref, w_ref, b_ref, o_ref):
    o_ref[...] = jnp.dot(x_ref[...], w_ref[...],
                         preferred_element_type=jnp.float32) + b_ref[...]


def mm_bias(x, w, b):
    n, Din = x.shape
    Fo = w.shape[1]
    return pl.pallas_call(
        _mm_bias_body,
        grid=(n // ROW_BLK,),
        in_specs=[
            pl.BlockSpec((ROW_BLK, Din), lambda i: (i, 0)),
            pl.BlockSpec((Din, Fo), lambda i: (0, 0)),
            pl.BlockSpec((1, Fo), lambda i: (0, 0)),
        ],
        out_specs=pl.BlockSpec((ROW_BLK, Fo), lambda i: (i, 0)),
        out_shape=jax.ShapeDtypeStruct((n, Fo), jnp.float32),
    )(x, w, b)


def _aggregate(hs, gidx, dst, coef):
    msg = hs.reshape(R * N, F)[gidx]
    return jnp.zeros((N, F), jnp.float32).at[dst].add(msg * coef[:, None])


def _bn_consts(p):
    s = p["bn_g"] / jnp.sqrt(p["bn_rv"] + 1e-5)
    c = p["bn_b"] + (p["b"] - p["bn_rm"]) * s
    return s.reshape(1, F), c.reshape(1, F), p["prelu"].reshape(1, 1)


def _conv_block(h, gidx, dst, coef, p, residual):
    w_all = jnp.concatenate([p["w_rel"], p["w_root"][None]], axis=0)
    hs6 = rel_matmul(h, w_all)
    agg = _aggregate(hs6[:R], gidx, dst, coef)
    s, c, a = _bn_consts(p)
    return conv_epilogue(agg, hs6[R], s, c, a, h if residual else None)


def kernel(x, edge_index, edge_attr, params):
    src = edge_index[0].astype(jnp.int32)
    dst = edge_index[1].astype(jnp.int32)
    et = edge_attr[:, 4].astype(jnp.int32)
    gidx = et * N + src

    cnt = jnp.zeros((N * R,), jnp.float32).at[dst * R + et].add(1.0)
    coef = 1.0 / jnp.maximum(cnt[dst * R + et], 1.0)

    h = _conv_block(x, gidx, dst, coef, params["enc0"], residual=False)
    h = _conv_block(h, gidx, dst, coef, params["enc1"], residual=True)

    wm = jnp.concatenate([params["fc_mu"]["w"], params["fc_logvar"]["w"]], axis=1)
    bm = jnp.concatenate([params["fc_mu"]["b"], params["fc_logvar"]["b"]])
    ml = mm_bias(h, wm, bm.reshape(1, -1))
    mu, logvar = ml[:, :64], ml[:, 64:]

    d = jnp.concatenate([mu, x], axis=1)
    d = _conv_block(d, gidx, dst, coef, params["dec0"], residual=False)
    d = _conv_block(d, gidx, dst, coef, params["dec1"], residual=True)
    out = mm_bias(d, params["fc_out"]["w"], params["fc_out"]["b"].reshape(1, -1))
    return (out, mu, logvar)


# speedup vs baseline: 8.4204x; 2.3588x over previous
import functools

import jax
import jax.numpy as jnp
from jax import lax
from jax.experimental import pallas as pl
from jax.experimental.pallas import tpu as pltpu
from jax.experimental.pallas import tpu_sc as plsc

N = 10000
E = 320000
F = 128
R = 5

ROW_BLK = 1000

NC = 2
NS = 16
L = 16
NW = NC * NS
SB = 128
NSB = E // SB
RPT = 624


def _rel_mm_body(x_ref, w_ref, o_ref):
    o_ref[0] = jnp.dot(x_ref[...], w_ref[0],
                       preferred_element_type=jnp.float32)


def rel_matmul(x, w_all):
    J, Din, Fo = w_all.shape
    n = x.shape[0]
    grid = (J, n // ROW_BLK)
    return pl.pallas_call(
        _rel_mm_body,
        grid=grid,
        in_specs=[
            pl.BlockSpec((ROW_BLK, Din), lambda j, i: (i, 0)),
            pl.BlockSpec((1, Din, Fo), lambda j, i: (j, 0, 0)),
        ],
        out_specs=pl.BlockSpec((1, ROW_BLK, Fo), lambda j, i: (j, i, 0)),
        out_shape=jax.ShapeDtypeStruct((J, n, Fo), jnp.float32),
    )(x, w_all)


def _epilogue_body(agg_ref, root_ref, s_ref, c_ref, a_ref, res_ref, o_ref):
    agg = agg_ref[0] + agg_ref[1]
    t = (agg + root_ref[...]) * s_ref[...] + c_ref[...]
    t = jnp.where(t >= 0, t, a_ref[0, 0] * t)
    o_ref[...] = t + res_ref[...]


def _epilogue_body_nores(agg_ref, root_ref, s_ref, c_ref, a_ref, o_ref):
    agg = agg_ref[0] + agg_ref[1]
    t = (agg + root_ref[...]) * s_ref[...] + c_ref[...]
    o_ref[...] = jnp.where(t >= 0, t, a_ref[0, 0] * t)


def conv_epilogue(agg2, root, s, c, a, res=None):
    n = root.shape[0]
    grid = (n // ROW_BLK,)
    agg_spec = pl.BlockSpec((NC, ROW_BLK, F), lambda i: (0, i, 0))
    row_spec = pl.BlockSpec((ROW_BLK, F), lambda i: (i, 0))
    vec_spec = pl.BlockSpec((1, F), lambda i: (0, 0))
    scal_spec = pl.BlockSpec((1, 1), lambda i: (0, 0))
    if res is None:
        return pl.pallas_call(
            _epilogue_body_nores,
            grid=grid,
            in_specs=[agg_spec, row_spec, vec_spec, vec_spec, scal_spec],
            out_specs=row_spec,
            out_shape=jax.ShapeDtypeStruct((n, F), jnp.float32),
        )(agg2, root, s, c, a)
    return pl.pallas_call(
        _epilogue_body,
        grid=grid,
        in_specs=[agg_spec, row_spec, vec_spec, vec_spec, scal_spec, row_spec],
        out_specs=row_spec,
        out_shape=jax.ShapeDtypeStruct((n, F), jnp.float32),
    )(agg2, root, s, c, a, res)


def _mm_bias_body(x_ref, w_ref, b_ref, o_ref):
    o_ref[...] = jnp.dot(x_ref[...], w_ref[...],
                         preferred_element_type=jnp.float32) + b_ref[...]


def mm_bias(x, w, b):
    n, Din = x.shape
    Fo = w.shape[1]
    return pl.pallas_call(
        _mm_bias_body,
        grid=(n // ROW_BLK,),
        in_specs=[
            pl.BlockSpec((ROW_BLK, Din), lambda i: (i, 0)),
            pl.BlockSpec((Din, Fo), lambda i: (0, 0)),
            pl.BlockSpec((1, Fo), lambda i: (0, 0)),
        ],
        out_specs=pl.BlockSpec((ROW_BLK, Fo), lambda i: (i, 0)),
        out_shape=jax.ShapeDtypeStruct((n, Fo), jnp.float32),
    )(x, w, b)


def _sc_agg_body(hs, gidx, dst, coef, out,
                 gidx_v, dst_v, coef_v, rows_v, agg_sh, sem):
    cid = lax.axis_index("c")
    sid = lax.axis_index("s")
    w = sid * NC + cid

    zero = jnp.zeros((L,), jnp.float32)

    def zero_body(r, carry):
        for g in range(F // L):
            rows_v[r, pl.ds(g * L, L)] = zero
        return carry

    lax.fori_loop(0, SB, zero_body, 0)

    base = sid * RPT
    for k in range(4):
        pltpu.sync_copy(rows_v.at[pl.ds(0, SB)],
                        agg_sh.at[pl.ds(base + k * SB, SB)])
    pltpu.sync_copy(rows_v.at[pl.ds(0, 112)],
                    agg_sh.at[pl.ds(base + 4 * SB, 112)])

    @pl.when(sid == NS - 1)
    def _zero_tail():
        pltpu.sync_copy(rows_v.at[pl.ds(0, 16)],
                        agg_sh.at[pl.ds(NS * RPT, 16)])

    plsc.subcore_barrier()

    trip = (NSB - w + NW - 1) // NW

    def edge_body(i, carry):
        eb = (w + i * NW) * SB
        pltpu.sync_copy(gidx.at[pl.ds(eb, SB)], gidx_v)
        pltpu.sync_copy(dst.at[pl.ds(eb, SB)], dst_v)
        pltpu.sync_copy(coef.at[pl.ds(eb, SB)], coef_v)
        pltpu.async_copy(hs.at[gidx_v], rows_v, sem).wait()

        def scale_body(j, c2):
            rb = j * L
            cvec = coef_v[pl.ds(rb, L)]
            for r in range(L):
                bvec = cvec.at[jnp.full((L,), r, jnp.int32)].get(
                    mode="promise_in_bounds")
                for g in range(F // L):
                    rows_v[rb + r, pl.ds(g * L, L)] = (
                        rows_v[rb + r, pl.ds(g * L, L)] * bvec)
            return c2

        lax.fori_loop(0, SB // L, scale_body, 0)
        pltpu.sync_copy(rows_v, agg_sh.at[dst_v], add=True)
        return carry

    lax.fori_loop(0, trip, edge_body, 0)
    plsc.subcore_barrier()

    for k in range(4):
        pltpu.sync_copy(agg_sh.at[pl.ds(base + k * SB, SB)],
                        out.at[cid, pl.ds(base + k * SB, SB), :])
    pltpu.sync_copy(agg_sh.at[pl.ds(base + 4 * SB, 112)],
                    out.at[cid, pl.ds(base + 4 * SB, 112), :])

    @pl.when(sid == NS - 1)
    def _out_tail():
        pltpu.sync_copy(agg_sh.at[pl.ds(NS * RPT, 16)],
                        out.at[cid, pl.ds(NS * RPT, 16), :])


def sc_aggregate(hs_flat, gidx, dst, coef):
    mesh = plsc.VectorSubcoreMesh(core_axis_name="c", subcore_axis_name="s",
                                  num_cores=NC, num_subcores=NS)
    return pl.kernel(
        _sc_agg_body,
        out_type=jax.ShapeDtypeStruct((NC, N, F), jnp.float32),
        mesh=mesh,
        scratch_types=[
            pltpu.VMEM((SB,), jnp.int32),
            pltpu.VMEM((SB,), jnp.int32),
            pltpu.VMEM((SB,), jnp.float32),
            pltpu.VMEM((SB, F), jnp.float32),
            pltpu.VMEM_SHARED((N, F), jnp.float32),
            pltpu.SemaphoreType.DMA,
        ],
    )(hs_flat, gidx, dst, coef)


def _bn_consts(p):
    s = p["bn_g"] / jnp.sqrt(p["bn_rv"] + 1e-5)
    c = p["bn_b"] + (p["b"] - p["bn_rm"]) * s
    return s.reshape(1, F), c.reshape(1, F), p["prelu"].reshape(1, 1)


def _conv_block(h, gidx, dst, coef, p, residual):
    w_all = jnp.concatenate([p["w_rel"], p["w_root"][None]], axis=0)
    hs6 = rel_matmul(h, w_all)
    agg2 = sc_aggregate(hs6.reshape((R + 1) * N, F), gidx, dst, coef)
    s, c, a = _bn_consts(p)
    return conv_epilogue(agg2, hs6[R], s, c, a, h if residual else None)


def kernel(x, edge_index, edge_attr, params):
    src = edge_index[0].astype(jnp.int32)
    dst = edge_index[1].astype(jnp.int32)
    et = edge_attr[:, 4].astype(jnp.int32)
    gidx = et * N + src

    cnt = jnp.zeros((N * R,), jnp.float32).at[dst * R + et].add(1.0)
    coef = 1.0 / jnp.maximum(cnt[dst * R + et], 1.0)

    h = _conv_block(x, gidx, dst, coef, params["enc0"], residual=False)
    h = _conv_block(h, gidx, dst, coef, params["enc1"], residual=True)

    wm = jnp.concatenate([params["fc_mu"]["w"], params["fc_logvar"]["w"]], axis=1)
    bm = jnp.concatenate([params["fc_mu"]["b"], params["fc_logvar"]["b"]])
    ml = mm_bias(h, wm, bm.reshape(1, -1))
    mu, logvar = ml[:, :64], ml[:, 64:]

    d = jnp.concatenate([mu, x], axis=1)
    d = _conv_block(d, gidx, dst, coef, params["dec0"], residual=False)
    d = _conv_block(d, gidx, dst, coef, params["dec1"], residual=True)
    out = mm_bias(d, params["fc_out"]["w"], params["fc_out"]["b"].reshape(1, -1))
    return (out, mu, logvar)
